# Initial kernel scaffold; baseline (speedup 1.0000x reference)
import jax
import jax.numpy as jnp
from jax import lax
from jax.experimental import pallas as pl
from jax.experimental.pallas import tpu as pltpu
from jax.experimental.pallas import tpu_sc as plsc

N = 10000
E = 320000
D = 128
EPS = 1e-5

NC = 2
NS = 16
L = 16
NW = NC * NS
NB = 79
EB = 128
EPT = NB * EB
EP = NW * EPT
NP = 10240
RPT = NP // NS
BNS = 1.0 / (1.0 + EPS) ** 0.5

_mesh = plsc.VectorSubcoreMesh(
    core_axis_name="c", subcore_axis_name="s", num_cores=NC, num_subcores=NS
)


def _deg_body(srcp, dstp, degs_out, degd_out, sidx, didx, degs, degd):
    c = lax.axis_index("c")
    s = lax.axis_index("s")
    wid = c * NS + s
    pltpu.sync_copy(srcp.at[wid], sidx)
    pltpu.sync_copy(dstp.at[wid], didx)
    zeros = jnp.zeros((L,), jnp.float32)

    def zero_body(i, _):
        degs[pl.ds(i * L, L)] = zeros
        degd[pl.ds(i * L, L)] = zeros
        return 0

    lax.fori_loop(0, NP // L, zero_body, 0)
    ones = jnp.ones((L,), jnp.float32)

    def cnt(j, _):
        for g in range(EB // L):
            plsc.addupdate_scatter(degs, [sidx[j, pl.ds(g * L, L)]], ones)
            plsc.addupdate_scatter(degd, [didx[j, pl.ds(g * L, L)]], ones)
        return 0

    lax.fori_loop(0, NB, cnt, 0)
    pltpu.sync_copy(degs, degs_out.at[wid])
    pltpu.sync_copy(degd, degd_out.at[wid])


_deg_call = pl.kernel(
    _deg_body,
    out_type=(
        jax.ShapeDtypeStruct((NW, NP), jnp.float32),
        jax.ShapeDtypeStruct((NW, NP), jnp.float32),
    ),
    mesh=_mesh,
    scratch_types=[
        pltpu.VMEM((NB, EB), jnp.int32),
        pltpu.VMEM((NB, EB), jnp.int32),
        pltpu.VMEM((NP,), jnp.float32),
        pltpu.VMEM((NP,), jnp.float32),
    ],
)


def _msg_body(table, srcp, dstp, ewp, ns, parts,
              sidx, didx, ewv, nsv, wbuf, rows, agg, sem):
    c = lax.axis_index("c")
    s = lax.axis_index("s")
    wid = c * NS + s
    pltpu.sync_copy(ns, nsv)
    pltpu.sync_copy(srcp.at[wid], sidx)
    pltpu.sync_copy(dstp.at[wid], didx)
    pltpu.sync_copy(ewp.at[wid], ewv)

    zeros = jnp.zeros((L,), jnp.float32)

    def zb(i, _):
        rows[i // 8, pl.ds((i % 8) * L, L)] = zeros
        return 0

    lax.fori_loop(0, EB * 8, zb, 0)

    def zs(k, _):
        pltpu.sync_copy(rows, agg.at[pl.ds(s * RPT + k * EB, EB)])
        return 0

    lax.fori_loop(0, RPT // EB, zs, 0)
    plsc.subcore_barrier()

    def blk(j, _):
        pltpu.async_copy(table.at[sidx.at[j]], rows, sem).wait()
        for g in range(EB // L):
            sl = pl.ds(g * L, L)
            nsg = plsc.load_gather(nsv, [sidx[j, sl]])
            wbuf[sl] = ewv[j, sl] * nsg

        def scale(e, _):
            w = wbuf[e]
            for cg in range(D // L):
                csl = pl.ds(cg * L, L)
                rows[e, csl] = rows[e, csl] * w
            return 0

        lax.fori_loop(0, EB, scale, 0)
        pltpu.sync_copy(rows, agg.at[didx.at[j]], add=True)
        return 0

    lax.fori_loop(0, NB, blk, 0)
    plsc.subcore_barrier()
    pltpu.sync_copy(agg.at[pl.ds(s * RPT, RPT)], parts.at[c, pl.ds(s * RPT, RPT)])


_msg_call = pl.kernel(
    _msg_body,
    out_type=jax.ShapeDtypeStruct((NC, NP, D), jnp.float32),
    mesh=_mesh,
    scratch_types=[
        pltpu.VMEM((NB, EB), jnp.int32),
        pltpu.VMEM((NB, EB), jnp.int32),
        pltpu.VMEM((NB, EB), jnp.float32),
        pltpu.VMEM((NP,), jnp.float32),
        pltpu.VMEM((EB,), jnp.float32),
        pltpu.VMEM((EB, D), jnp.float32),
        pltpu.VMEM_SHARED((NP, D), jnp.float32),
        pltpu.SemaphoreType.DMA,
    ],
)


def _norm_body(ds_ref, dd_ref, ns_ref, nd_ref):
    degs = jnp.sum(ds_ref[...], axis=0)
    degd = jnp.sum(dd_ref[...], axis=0)
    ns_ref[...] = lax.rsqrt(jnp.maximum(degs, 1.0))
    nd_ref[...] = lax.rsqrt(jnp.maximum(degd, 1.0))


_norm_call = pl.pallas_call(
    _norm_body,
    out_shape=(
        jax.ShapeDtypeStruct((NP // 128, 128), jnp.float32),
        jax.ShapeDtypeStruct((NP // 128, 128), jnp.float32),
    ),
)


def _make_dense(with_res):
    def body(*refs):
        if with_res:
            parts_ref, nd_ref, w_ref, b_ref, g_ref, be_ref, res_ref, out_ref = refs
        else:
            parts_ref, nd_ref, w_ref, b_ref, g_ref, be_ref, out_ref = refs
        p = (parts_ref[0] + parts_ref[1]) * nd_ref[...]
        acc = jnp.dot(p, w_ref[...], preferred_element_type=jnp.float32)
        y = g_ref[...] * ((acc + b_ref[...]) * BNS) + be_ref[...]
        if with_res:
            y = y + res_ref[...]
        out_ref[...] = y

    R = 512
    in_specs = [
        pl.BlockSpec((2, R, D), lambda i: (0, i, 0)),
        pl.BlockSpec((R, 1), lambda i: (i, 0)),
        pl.BlockSpec((D, D), lambda i: (0, 0)),
        pl.BlockSpec((1, D), lambda i: (0, 0)),
        pl.BlockSpec((1, D), lambda i: (0, 0)),
        pl.BlockSpec((1, D), lambda i: (0, 0)),
    ]
    if with_res:
        in_specs.append(pl.BlockSpec((R, D), lambda i: (i, 0)))
    return pl.pallas_call(
        body,
        grid=(NP // R,),
        in_specs=in_specs,
        out_specs=pl.BlockSpec((R, D), lambda i: (i, 0)),
        out_shape=jax.ShapeDtypeStruct((NP, D), jnp.float32),
    )


_dense0 = _make_dense(False)
_dense1 = _make_dense(True)


def kernel(x, edge_index, edge_weight, W1, b1, g1, be1, W2, b2, g2, be2):
    src = edge_index[0]
    dst = edge_index[1]
    pad = EP - E
    srcp = jnp.concatenate([src, jnp.full((pad,), N, jnp.int32)]).reshape(NW, NB, EB)
    dstp = jnp.concatenate([dst, jnp.full((pad,), N, jnp.int32)]).reshape(NW, NB, EB)
    ewp = jnp.concatenate(
        [edge_weight, jnp.zeros((pad,), jnp.float32)]
    ).reshape(NW, NB, EB)
    xp = jnp.zeros((NP, D), jnp.float32).at[:N].set(x)

    degs_p, degd_p = _deg_call(srcp, dstp)
    ns80, nd80 = _norm_call(
        degs_p.reshape(NW, NP // 128, 128), degd_p.reshape(NW, NP // 128, 128)
    )
    ns = ns80.reshape(NP)
    ndcol = nd80.reshape(NP, 1)

    parts1 = _msg_call(xp, srcp, dstp, ewp, ns)
    h1 = _dense0(parts1, ndcol, W1, b1.reshape(1, D), g1.reshape(1, D),
                 be1.reshape(1, D))
    parts2 = _msg_call(h1, srcp, dstp, ewp, ns)
    out = _dense1(parts2, ndcol, W2, b2.reshape(1, D), g2.reshape(1, D),
                  be2.reshape(1, D), xp)
    return out[:N]


# speedup vs baseline: 3.0127x; 3.0127x over previous
import jax
import jax.numpy as jnp
from jax import lax
from jax.experimental import pallas as pl
from jax.experimental.pallas import tpu as pltpu
from jax.experimental.pallas import tpu_sc as plsc

N = 10000
E = 320000
D = 128
EPS = 1e-5

NC = 2
NS = 16
L = 16
NW = NC * NS
EB = 128
NB = 79
NB2 = 158
EP = NW * NB * EB
NP = 10240
RPT = NP // NS
CD = D // NC
BNS = 1.0 / (1.0 + EPS) ** 0.5

_mesh = plsc.VectorSubcoreMesh(
    core_axis_name="c", subcore_axis_name="s", num_cores=NC, num_subcores=NS
)
_sc_params = pltpu.CompilerParams(
    needs_layout_passes=False, use_tc_tiling_on_sc=False
)


def _deg_body(srcp, dstp, degs_out, degd_out, sidx, didx, degs, degd):
    c = lax.axis_index("c")
    s = lax.axis_index("s")
    wid = c * NS + s
    pltpu.sync_copy(srcp.at[wid], sidx)
    pltpu.sync_copy(dstp.at[wid], didx)
    zeros = jnp.zeros((L,), jnp.float32)

    def zero_body(i, _):
        degs[pl.ds(i * L, L)] = zeros
        degd[pl.ds(i * L, L)] = zeros
        return 0

    lax.fori_loop(0, NP // L, zero_body, 0)
    ones = jnp.ones((L,), jnp.float32)

    def cnt(j, _):
        for g in range(EB // L):
            plsc.addupdate_scatter(degs, [sidx[j, pl.ds(g * L, L)]], ones)
            plsc.addupdate_scatter(degd, [didx[j, pl.ds(g * L, L)]], ones)
        return 0

    lax.fori_loop(0, NB, cnt, 0)
    pltpu.sync_copy(degs, degs_out.at[wid])
    pltpu.sync_copy(degd, degd_out.at[wid])


_deg_call = pl.kernel(
    _deg_body,
    out_type=(
        jax.ShapeDtypeStruct((NW, NP), jnp.float32),
        jax.ShapeDtypeStruct((NW, NP), jnp.float32),
    ),
    mesh=_mesh,
    scratch_types=[
        pltpu.VMEM((NB, EB), jnp.int32),
        pltpu.VMEM((NB, EB), jnp.int32),
        pltpu.VMEM((NP,), jnp.float32),
        pltpu.VMEM((NP,), jnp.float32),
    ],
    compiler_params=_sc_params,
)


def _msg_body(table, srcp, dstp, ewp, ns2, parts,
              sidx, didx, ewv, nsv, wbuf, rows, agg, sem):
    c = lax.axis_index("c")
    s = lax.axis_index("s")
    pltpu.sync_copy(ns2, nsv)
    pltpu.sync_copy(srcp.at[s], sidx)
    pltpu.sync_copy(dstp.at[s], didx)
    pltpu.sync_copy(ewp.at[s], ewv)

    coff = c * NP

    def off(j, _):
        for g in range(EB // L):
            sl = pl.ds(g * L, L)
            sidx[j, sl] = sidx[j, sl] + coff
        return 0

    lax.fori_loop(0, NB2, off, 0)

    zeros = jnp.zeros((L,), jnp.float32)

    def zb(i, _):
        rows[i // (CD // L), pl.ds((i % (CD // L)) * L, L)] = zeros
        return 0

    lax.fori_loop(0, EB * (CD // L), zb, 0)

    def zs(k, _):
        pltpu.sync_copy(rows, agg.at[pl.ds(s * RPT + k * EB, EB)])
        return 0

    lax.fori_loop(0, RPT // EB, zs, 0)
    plsc.subcore_barrier()

    def blk(j, _):
        pltpu.async_copy(table.at[sidx.at[j]], rows, sem).wait()
        for g in range(EB // L):
            sl = pl.ds(g * L, L)
            nsg = plsc.load_gather(nsv, [sidx[j, sl]])
            wbuf[sl] = ewv[j, sl] * nsg

        def scale(g2, _):
            wv = wbuf[pl.ds(g2 * L, L)]
            for k in range(L):
                w = wv[k]
                e = g2 * L + k
                for cg in range(CD // L):
                    csl = pl.ds(cg * L, L)
                    rows[e, csl] = rows[e, csl] * w
            return 0

        lax.fori_loop(0, EB // L, scale, 0)
        pltpu.sync_copy(rows, agg.at[didx.at[j]], add=True)
        return 0

    lax.fori_loop(0, NB2, blk, 0)
    plsc.subcore_barrier()
    pltpu.sync_copy(agg.at[pl.ds(s * RPT, RPT)], parts.at[c, pl.ds(s * RPT, RPT)])


_msg_call = pl.kernel(
    _msg_body,
    out_type=jax.ShapeDtypeStruct((NC, NP, CD), jnp.float32),
    mesh=_mesh,
    scratch_types=[
        pltpu.VMEM((NB2, EB), jnp.int32),
        pltpu.VMEM((NB2, EB), jnp.int32),
        pltpu.VMEM((NB2, EB), jnp.float32),
        pltpu.VMEM((NC * NP,), jnp.float32),
        pltpu.VMEM((EB,), jnp.float32),
        pltpu.VMEM((EB, CD), jnp.float32),
        pltpu.VMEM_SHARED((NP, CD), jnp.float32),
        pltpu.SemaphoreType.DMA,
    ],
    compiler_params=_sc_params,
)


def _norm_body(ds_ref, dd_ref, ns_ref, nd_ref):
    degs = jnp.sum(ds_ref[...], axis=0)
    degd = jnp.sum(dd_ref[...], axis=0)
    ns_ref[...] = lax.rsqrt(jnp.maximum(degs, 1.0))
    nd_ref[...] = lax.rsqrt(jnp.maximum(degd, 1.0))


_norm_call = pl.pallas_call(
    _norm_body,
    out_shape=(
        jax.ShapeDtypeStruct((NP // 128, 128), jnp.float32),
        jax.ShapeDtypeStruct((NP // 128, 128), jnp.float32),
    ),
)


def _make_dense(with_res):
    def body(*refs):
        if with_res:
            parts_ref, nd_ref, w_ref, b_ref, g_ref, be_ref, res_ref, out_ref = refs
        else:
            parts_ref, nd_ref, w_ref, b_ref, g_ref, be_ref, out_ref = refs
        p = jnp.concatenate([parts_ref[0], parts_ref[1]], axis=1) * nd_ref[...]
        acc = jnp.dot(p, w_ref[...], preferred_element_type=jnp.float32)
        y = g_ref[...] * ((acc + b_ref[...]) * BNS) + be_ref[...]
        if with_res:
            y = y + jnp.concatenate([res_ref[0], res_ref[1]], axis=1)
        out_ref[0] = y[:, :CD]
        out_ref[1] = y[:, CD:]

    R = 512
    in_specs = [
        pl.BlockSpec((NC, R, CD), lambda i: (0, i, 0)),
        pl.BlockSpec((R, 1), lambda i: (i, 0)),
        pl.BlockSpec((D, D), lambda i: (0, 0)),
        pl.BlockSpec((1, D), lambda i: (0, 0)),
        pl.BlockSpec((1, D), lambda i: (0, 0)),
        pl.BlockSpec((1, D), lambda i: (0, 0)),
    ]
    if with_res:
        in_specs.append(pl.BlockSpec((NC, R, CD), lambda i: (0, i, 0)))
    return pl.pallas_call(
        body,
        grid=(NP // R,),
        in_specs=in_specs,
        out_specs=pl.BlockSpec((NC, R, CD), lambda i: (0, i, 0)),
        out_shape=jax.ShapeDtypeStruct((NC, NP, CD), jnp.float32),
    )


_dense0 = _make_dense(False)
_dense1 = _make_dense(True)


def kernel(x, edge_index, edge_weight, W1, b1, g1, be1, W2, b2, g2, be2):
    src = edge_index[0]
    dst = edge_index[1]
    pad = EP - E
    srcf = jnp.concatenate([src, jnp.full((pad,), N, jnp.int32)])
    dstf = jnp.concatenate([dst, jnp.full((pad,), N, jnp.int32)])
    ewf = jnp.concatenate([edge_weight, jnp.zeros((pad,), jnp.float32)])
    srcp32 = srcf.reshape(NW, NB, EB)
    dstp32 = dstf.reshape(NW, NB, EB)
    srcp16 = srcf.reshape(NS, NB2, EB)
    dstp16 = dstf.reshape(NS, NB2, EB)
    ewp16 = ewf.reshape(NS, NB2, EB)

    xp = jnp.zeros((NP, D), jnp.float32).at[:N].set(x)
    xs = jnp.stack([xp[:, :CD], xp[:, CD:]])

    degs_p, degd_p = _deg_call(srcp32, dstp32)
    ns80, nd80 = _norm_call(
        degs_p.reshape(NW, NP // 128, 128), degd_p.reshape(NW, NP // 128, 128)
    )
    ns2 = jnp.tile(ns80.reshape(NP), NC)
    ndcol = nd80.reshape(NP, 1)

    b1r, g1r, be1r = b1.reshape(1, D), g1.reshape(1, D), be1.reshape(1, D)
    b2r, g2r, be2r = b2.reshape(1, D), g2.reshape(1, D), be2.reshape(1, D)

    parts1 = _msg_call(xs.reshape(NC * NP, CD), srcp16, dstp16, ewp16, ns2)
    h1 = _dense0(parts1, ndcol, W1, b1r, g1r, be1r)
    parts2 = _msg_call(h1.reshape(NC * NP, CD), srcp16, dstp16, ewp16, ns2)
    out = _dense1(parts2, ndcol, W2, b2r, g2r, be2r, xs)
    return jnp.concatenate([out[0, :N], out[1, :N]], axis=1)


# speedup vs baseline: 5.7706x; 1.9154x over previous
import jax
import jax.numpy as jnp
from jax import lax
from jax.experimental import pallas as pl
from jax.experimental.pallas import tpu as pltpu
from jax.experimental.pallas import tpu_sc as plsc

N = 10000
E = 320000
D = 128
EPS = 1e-5

NC = 2
NS = 16
L = 16
NW = NC * NS
EB = 128
NB = 79
NB2 = 158
EP = NW * NB * EB
NP = 10240
RPT = NP // NS
CD = D // NC
BNS = 1.0 / (1.0 + EPS) ** 0.5

_mesh = plsc.VectorSubcoreMesh(
    core_axis_name="c", subcore_axis_name="s", num_cores=NC, num_subcores=NS
)
_sc_params = pltpu.CompilerParams(
    needs_layout_passes=False, use_tc_tiling_on_sc=False
)


def _deg_body(srcp, dstp, degs_out, degd_out, sidx, didx, degs, degd):
    c = lax.axis_index("c")
    s = lax.axis_index("s")
    wid = c * NS + s
    pltpu.sync_copy(srcp.at[wid], sidx)
    pltpu.sync_copy(dstp.at[wid], didx)
    zeros = jnp.zeros((L,), jnp.float32)

    def zero_body(i, _):
        degs[pl.ds(i * L, L)] = zeros
        degd[pl.ds(i * L, L)] = zeros
        return 0

    lax.fori_loop(0, NP // L, zero_body, 0)
    ones = jnp.ones((L,), jnp.float32)

    def cnt(j, _):
        for g in range(EB // L):
            plsc.addupdate_scatter(degs, [sidx[j, pl.ds(g * L, L)]], ones)
            plsc.addupdate_scatter(degd, [didx[j, pl.ds(g * L, L)]], ones)
        return 0

    lax.fori_loop(0, NB, cnt, 0)
    pltpu.sync_copy(degs, degs_out.at[wid])
    pltpu.sync_copy(degd, degd_out.at[wid])


_deg_call = pl.kernel(
    _deg_body,
    out_type=(
        jax.ShapeDtypeStruct((NW, NP), jnp.float32),
        jax.ShapeDtypeStruct((NW, NP), jnp.float32),
    ),
    mesh=_mesh,
    scratch_types=[
        pltpu.VMEM((NB, EB), jnp.int32),
        pltpu.VMEM((NB, EB), jnp.int32),
        pltpu.VMEM((NP,), jnp.float32),
        pltpu.VMEM((NP,), jnp.float32),
    ],
    compiler_params=_sc_params,
)


def _msg_body(table, srcp, dstp, ewp, ns, parts,
              sidx, didx, ewv, nsv, rows0, rows1, agg,
              gsem0, gsem1, ssem0, ssem1):
    c = lax.axis_index("c")
    s = lax.axis_index("s")
    pltpu.sync_copy(ns, nsv)
    pltpu.sync_copy(srcp.at[s], sidx)
    pltpu.sync_copy(dstp.at[s], didx)
    pltpu.sync_copy(ewp.at[s], ewv)

    coff = c * NP

    def off(j, _):
        for g in range(EB // L):
            sl = pl.ds(g * L, L)
            idx = sidx[j, sl]
            ewv[j, sl] = ewv[j, sl] * plsc.load_gather(nsv, [idx])
            sidx[j, sl] = idx + coff
        return 0

    lax.fori_loop(0, NB2, off, 0)

    zeros = jnp.zeros((L,), jnp.float32)

    def zb(i, _):
        rows0[i // (CD // L), pl.ds((i % (CD // L)) * L, L)] = zeros
        return 0

    lax.fori_loop(0, EB * (CD // L), zb, 0)

    def zs(k, _):
        pltpu.sync_copy(rows0, agg.at[pl.ds(s * RPT + k * EB, EB)])
        return 0

    lax.fori_loop(0, RPT // EB, zs, 0)
    plsc.subcore_barrier()

    def gather(j, buf, sem):
        return pltpu.async_copy(table.at[sidx.at[j]], buf, sem)

    def gather_wait(j, buf, sem):
        pltpu.make_async_copy(table.at[sidx.at[j]], buf, sem).wait()

    def scatter(j, buf, sem):
        return pltpu.async_copy(buf, agg.at[didx.at[j]], sem, add=True)

    def scatter_wait(j, buf, sem):
        pltpu.make_async_copy(buf, agg.at[didx.at[j]], sem).wait()

    def scale(j, buf):
        def sc_body(g2, _):
            wv = ewv[j, pl.ds(g2 * L, L)]
            for k in range(L):
                w = wv[k]
                e = g2 * L + k
                for cg in range(CD // L):
                    csl = pl.ds(cg * L, L)
                    buf[e, csl] = buf[e, csl] * w
            return 0

        lax.fori_loop(0, EB // L, sc_body, 0)

    gather(0, rows0, gsem0)

    def blk2(j2, _):
        j0 = j2 * 2
        j1 = j0 + 1
        gather_wait(j0, rows0, gsem0)

        @pl.when(j2 >= 1)
        def _():
            scatter_wait(j1 - 2, rows1, ssem1)

        gather(j1, rows1, gsem1)
        scale(j0, rows0)
        scatter(j0, rows0, ssem0)
        gather_wait(j1, rows1, gsem1)
        scale(j1, rows1)
        scatter_wait(j0, rows0, ssem0)

        @pl.when(j2 + 1 < NB2 // 2)
        def _():
            gather(j0 + 2, rows0, gsem0)

        scatter(j1, rows1, ssem1)
        return 0

    lax.fori_loop(0, NB2 // 2, blk2, 0)
    scatter_wait(NB2 - 1, rows1, ssem1)
    plsc.subcore_barrier()
    pltpu.sync_copy(agg.at[pl.ds(s * RPT, RPT)], parts.at[c, pl.ds(s * RPT, RPT)])


_msg_call = pl.kernel(
    _msg_body,
    out_type=jax.ShapeDtypeStruct((NC, NP, CD), jnp.float32),
    mesh=_mesh,
    scratch_types=[
        pltpu.VMEM((NB2, EB), jnp.int32),
        pltpu.VMEM((NB2, EB), jnp.int32),
        pltpu.VMEM((NB2, EB), jnp.float32),
        pltpu.VMEM((NP,), jnp.float32),
        pltpu.VMEM((EB, CD), jnp.float32),
        pltpu.VMEM((EB, CD), jnp.float32),
        pltpu.VMEM_SHARED((NP, CD), jnp.float32),
        pltpu.SemaphoreType.DMA,
        pltpu.SemaphoreType.DMA,
        pltpu.SemaphoreType.DMA,
        pltpu.SemaphoreType.DMA,
    ],
    compiler_params=_sc_params,
)


def _norm_body(ds_ref, dd_ref, ns_ref, nd_ref):
    degs = jnp.sum(ds_ref[...], axis=0)
    degd = jnp.sum(dd_ref[...], axis=0)
    ns_ref[...] = lax.rsqrt(jnp.maximum(degs, 1.0))
    nd_ref[...] = lax.rsqrt(jnp.maximum(degd, 1.0))


_norm_call = pl.pallas_call(
    _norm_body,
    out_shape=(
        jax.ShapeDtypeStruct((NP // 128, 128), jnp.float32),
        jax.ShapeDtypeStruct((NP // 128, 128), jnp.float32),
    ),
)


def _make_dense(with_res):
    def body(*refs):
        if with_res:
            parts_ref, nd_ref, w_ref, b_ref, g_ref, be_ref, res_ref, out_ref = refs
        else:
            parts_ref, nd_ref, w_ref, b_ref, g_ref, be_ref, out_ref = refs
        p = jnp.concatenate([parts_ref[0], parts_ref[1]], axis=1) * nd_ref[...]
        acc = jnp.dot(p, w_ref[...], preferred_element_type=jnp.float32)
        y = g_ref[...] * ((acc + b_ref[...]) * BNS) + be_ref[...]
        if with_res:
            y = y + jnp.concatenate([res_ref[0], res_ref[1]], axis=1)
        out_ref[0] = y[:, :CD]
        out_ref[1] = y[:, CD:]

    R = 512
    in_specs = [
        pl.BlockSpec((NC, R, CD), lambda i: (0, i, 0)),
        pl.BlockSpec((R, 1), lambda i: (i, 0)),
        pl.BlockSpec((D, D), lambda i: (0, 0)),
        pl.BlockSpec((1, D), lambda i: (0, 0)),
        pl.BlockSpec((1, D), lambda i: (0, 0)),
        pl.BlockSpec((1, D), lambda i: (0, 0)),
    ]
    if with_res:
        in_specs.append(pl.BlockSpec((NC, R, CD), lambda i: (0, i, 0)))
    return pl.pallas_call(
        body,
        grid=(NP // R,),
        in_specs=in_specs,
        out_specs=pl.BlockSpec((NC, R, CD), lambda i: (0, i, 0)),
        out_shape=jax.ShapeDtypeStruct((NC, NP, CD), jnp.float32),
    )


_dense0 = _make_dense(False)
_dense1 = _make_dense(True)


def kernel(x, edge_index, edge_weight, W1, b1, g1, be1, W2, b2, g2, be2):
    src = edge_index[0]
    dst = edge_index[1]
    pad = EP - E
    srcf = jnp.concatenate([src, jnp.full((pad,), N, jnp.int32)])
    dstf = jnp.concatenate([dst, jnp.full((pad,), N, jnp.int32)])
    ewf = jnp.concatenate([edge_weight, jnp.zeros((pad,), jnp.float32)])
    srcp32 = srcf.reshape(NW, NB, EB)
    dstp32 = dstf.reshape(NW, NB, EB)
    srcp16 = srcf.reshape(NS, NB2, EB)
    dstp16 = dstf.reshape(NS, NB2, EB)
    ewp16 = ewf.reshape(NS, NB2, EB)

    xp = jnp.zeros((NP, D), jnp.float32).at[:N].set(x)
    xs = jnp.stack([xp[:, :CD], xp[:, CD:]])

    degs_p, degd_p = _deg_call(srcp32, dstp32)
    ns80, nd80 = _norm_call(
        degs_p.reshape(NW, NP // 128, 128), degd_p.reshape(NW, NP // 128, 128)
    )
    ns = ns80.reshape(NP)
    ndcol = nd80.reshape(NP, 1)

    b1r, g1r, be1r = b1.reshape(1, D), g1.reshape(1, D), be1.reshape(1, D)
    b2r, g2r, be2r = b2.reshape(1, D), g2.reshape(1, D), be2.reshape(1, D)

    parts1 = _msg_call(xs.reshape(NC * NP, CD), srcp16, dstp16, ewp16, ns)
    h1 = _dense0(parts1, ndcol, W1, b1r, g1r, be1r)
    parts2 = _msg_call(h1.reshape(NC * NP, CD), srcp16, dstp16, ewp16, ns)
    out = _dense1(parts2, ndcol, W2, b2r, g2r, be2r, xs)
    return jnp.concatenate([out[0, :N], out[1, :N]], axis=1)


# speedup vs baseline: 5.8509x; 1.0139x over previous
import jax
import jax.numpy as jnp
from jax import lax
from jax.experimental import pallas as pl
from jax.experimental.pallas import tpu as pltpu
from jax.experimental.pallas import tpu_sc as plsc

N = 10000
E = 320000
D = 128
EPS = 1e-5

NC = 2
NS = 16
L = 16
NW = NC * NS
EB = 128
NB = 79
NB3 = 159
EP = NW * NB * EB
EP2 = NS * NB3 * EB
NP = 10240
RPT = NP // NS
CD = D // NC
BNS = 1.0 / (1.0 + EPS) ** 0.5

_mesh = plsc.VectorSubcoreMesh(
    core_axis_name="c", subcore_axis_name="s", num_cores=NC, num_subcores=NS
)
_sc_params = pltpu.CompilerParams(
    needs_layout_passes=False, use_tc_tiling_on_sc=False
)


def _deg_body(srcp, dstp, degs_out, degd_out, sidx, didx, degs, degd):
    c = lax.axis_index("c")
    s = lax.axis_index("s")
    wid = c * NS + s
    pltpu.sync_copy(srcp.at[wid], sidx)
    pltpu.sync_copy(dstp.at[wid], didx)
    zeros = jnp.zeros((L,), jnp.float32)

    def zero_body(i, _):
        degs[pl.ds(i * L, L)] = zeros
        degd[pl.ds(i * L, L)] = zeros
        return 0

    lax.fori_loop(0, NP // L, zero_body, 0)
    ones = jnp.ones((L,), jnp.float32)

    def cnt(j, _):
        for g in range(EB // L):
            plsc.addupdate_scatter(degs, [sidx[j, pl.ds(g * L, L)]], ones)
            plsc.addupdate_scatter(degd, [didx[j, pl.ds(g * L, L)]], ones)
        return 0

    lax.fori_loop(0, NB, cnt, 0)
    pltpu.sync_copy(degs, degs_out.at[wid])
    pltpu.sync_copy(degd, degd_out.at[wid])


_deg_call = pl.kernel(
    _deg_body,
    out_type=(
        jax.ShapeDtypeStruct((NW, NP), jnp.float32),
        jax.ShapeDtypeStruct((NW, NP), jnp.float32),
    ),
    mesh=_mesh,
    scratch_types=[
        pltpu.VMEM((NB, EB), jnp.int32),
        pltpu.VMEM((NB, EB), jnp.int32),
        pltpu.VMEM((NP,), jnp.float32),
        pltpu.VMEM((NP,), jnp.float32),
    ],
    compiler_params=_sc_params,
)


def _msg_body(table, srcp, dstp, ewp, parts,
              sidx, didx, ewv, rows0, rows1, rows2, agg,
              gsem0, gsem1, gsem2, ssem0, ssem1, ssem2):
    c = lax.axis_index("c")
    s = lax.axis_index("s")
    pltpu.sync_copy(srcp.at[s], sidx)
    pltpu.sync_copy(dstp.at[s], didx)
    pltpu.sync_copy(ewp.at[s], ewv)

    coff = c * NP

    def off(j, _):
        for g in range(EB // L):
            sl = pl.ds(g * L, L)
            sidx[j, sl] = sidx[j, sl] + coff
        return 0

    lax.fori_loop(0, NB3, off, 0)

    zeros = jnp.zeros((L,), jnp.float32)

    def zb(i, _):
        rows0[i // (CD // L), pl.ds((i % (CD // L)) * L, L)] = zeros
        return 0

    lax.fori_loop(0, EB * (CD // L), zb, 0)

    def zs(k, _):
        pltpu.sync_copy(rows0, agg.at[pl.ds(s * RPT + k * EB, EB)])
        return 0

    lax.fori_loop(0, RPT // EB, zs, 0)
    plsc.subcore_barrier()

    def gather(j, buf, sem):
        return pltpu.async_copy(table.at[sidx.at[j]], buf, sem)

    def gather_wait(j, buf, sem):
        pltpu.make_async_copy(table.at[sidx.at[j]], buf, sem).wait()

    def scatter(j, buf, sem):
        return pltpu.async_copy(buf, agg.at[didx.at[j]], sem, add=True)

    def scatter_wait(j, buf, sem):
        pltpu.make_async_copy(buf, agg.at[didx.at[j]], sem).wait()

    def scale(j, buf):
        def sc_body(g2, _):
            wv = ewv[j, pl.ds(g2 * L, L)]
            for k in range(L):
                w = wv[k]
                e = g2 * L + k
                for cg in range(CD // L):
                    csl = pl.ds(cg * L, L)
                    buf[e, csl] = buf[e, csl] * w
            return 0

        lax.fori_loop(0, EB // L, sc_body, 0)

    bufs = (rows0, rows1, rows2)
    gsems = (gsem0, gsem1, gsem2)
    ssems = (ssem0, ssem1, ssem2)
    gather(0, rows0, gsem0)
    gather(1, rows1, gsem1)

    def blk3(j3, _):
        for t in range(3):
            j = j3 * 3 + t
            buf = bufs[t]
            pre = bufs[(t + 2) % 3]
            psem = (t + 2) % 3
            gather_wait(j, buf, gsems[t])
            scale(j, buf)
            if t == 0:
                @pl.when(j3 >= 1)
                def _():
                    scatter_wait(j - 1, pre, ssems[psem])

                gather(j + 2, pre, gsems[psem])
            else:
                scatter_wait(j - 1, pre, ssems[psem])

                @pl.when(j3 < NB3 // 3 - 1)
                def _():
                    gather(j + 2, pre, gsems[psem])

            scatter(j, buf, ssems[t])
        return 0

    lax.fori_loop(0, NB3 // 3, blk3, 0)
    scatter_wait(NB3 - 1, rows2, ssem2)
    plsc.subcore_barrier()
    pltpu.sync_copy(agg.at[pl.ds(s * RPT, RPT)], parts.at[c, pl.ds(s * RPT, RPT)])


_msg_call = pl.kernel(
    _msg_body,
    out_type=jax.ShapeDtypeStruct((NC, NP, CD), jnp.float32),
    mesh=_mesh,
    scratch_types=[
        pltpu.VMEM((NB3, EB), jnp.int32),
        pltpu.VMEM((NB3, EB), jnp.int32),
        pltpu.VMEM((NB3, EB), jnp.float32),
        pltpu.VMEM((EB, CD), jnp.float32),
        pltpu.VMEM((EB, CD), jnp.float32),
        pltpu.VMEM((EB, CD), jnp.float32),
        pltpu.VMEM_SHARED((NP, CD), jnp.float32),
        pltpu.SemaphoreType.DMA,
        pltpu.SemaphoreType.DMA,
        pltpu.SemaphoreType.DMA,
        pltpu.SemaphoreType.DMA,
        pltpu.SemaphoreType.DMA,
        pltpu.SemaphoreType.DMA,
    ],
    compiler_params=_sc_params,
)


def _scalet_body(xs_ref, ns_ref, out_ref):
    out_ref[...] = xs_ref[...] * ns_ref[...]


_scale_table = pl.pallas_call(
    _scalet_body,
    grid=(NP // 512,),
    in_specs=[
        pl.BlockSpec((NC, 512, CD), lambda i: (0, i, 0)),
        pl.BlockSpec((512, 1), lambda i: (i, 0)),
    ],
    out_specs=pl.BlockSpec((NC, 512, CD), lambda i: (0, i, 0)),
    out_shape=jax.ShapeDtypeStruct((NC, NP, CD), jnp.float32),
)


def _norm_body(ds_ref, dd_ref, ns_ref, nd_ref):
    degs = jnp.sum(ds_ref[...], axis=0)
    degd = jnp.sum(dd_ref[...], axis=0)
    ns_ref[...] = lax.rsqrt(jnp.maximum(degs, 1.0))
    nd_ref[...] = lax.rsqrt(jnp.maximum(degd, 1.0))


_norm_call = pl.pallas_call(
    _norm_body,
    out_shape=(
        jax.ShapeDtypeStruct((NP // 128, 128), jnp.float32),
        jax.ShapeDtypeStruct((NP // 128, 128), jnp.float32),
    ),
)


def _make_dense(with_res, ns_scale):
    def body(*refs):
        refs = list(refs)
        parts_ref = refs.pop(0)
        nd_ref = refs.pop(0)
        ns_ref = refs.pop(0) if ns_scale else None
        w_ref, b_ref, g_ref, be_ref = refs[:4]
        refs = refs[4:]
        res_ref = refs.pop(0) if with_res else None
        out_ref = refs.pop(0)
        p = jnp.concatenate([parts_ref[0], parts_ref[1]], axis=1) * nd_ref[...]
        acc = jnp.dot(p, w_ref[...], preferred_element_type=jnp.float32)
        y = g_ref[...] * ((acc + b_ref[...]) * BNS) + be_ref[...]
        if with_res:
            y = y + jnp.concatenate([res_ref[0], res_ref[1]], axis=1)
        if ns_scale:
            y = y * ns_ref[...]
        out_ref[0] = y[:, :CD]
        out_ref[1] = y[:, CD:]

    R = 512
    in_specs = [
        pl.BlockSpec((NC, R, CD), lambda i: (0, i, 0)),
        pl.BlockSpec((R, 1), lambda i: (i, 0)),
    ]
    if ns_scale:
        in_specs.append(pl.BlockSpec((R, 1), lambda i: (i, 0)))
    in_specs += [
        pl.BlockSpec((D, D), lambda i: (0, 0)),
        pl.BlockSpec((1, D), lambda i: (0, 0)),
        pl.BlockSpec((1, D), lambda i: (0, 0)),
        pl.BlockSpec((1, D), lambda i: (0, 0)),
    ]
    if with_res:
        in_specs.append(pl.BlockSpec((NC, R, CD), lambda i: (0, i, 0)))
    return pl.pallas_call(
        body,
        grid=(NP // R,),
        in_specs=in_specs,
        out_specs=pl.BlockSpec((NC, R, CD), lambda i: (0, i, 0)),
        out_shape=jax.ShapeDtypeStruct((NC, NP, CD), jnp.float32),
    )


_dense_mid = _make_dense(False, True)
_dense_final = _make_dense(True, False)


def kernel(x, edge_index, edge_weight, W1, b1, g1, be1, W2, b2, g2, be2):
    src = edge_index[0]
    dst = edge_index[1]
    pad = EP - E
    pad2 = EP2 - E
    srcp32 = jnp.concatenate(
        [src, jnp.full((pad,), N, jnp.int32)]).reshape(NW, NB, EB)
    dstp32 = jnp.concatenate(
        [dst, jnp.full((pad,), N, jnp.int32)]).reshape(NW, NB, EB)
    srcp16 = jnp.concatenate(
        [src, jnp.full((pad2,), N, jnp.int32)]).reshape(NS, NB3, EB)
    dstp16 = jnp.concatenate(
        [dst, jnp.full((pad2,), N, jnp.int32)]).reshape(NS, NB3, EB)
    ewp16 = jnp.concatenate(
        [edge_weight, jnp.zeros((pad2,), jnp.float32)]).reshape(NS, NB3, EB)

    xp = jnp.zeros((NP, D), jnp.float32).at[:N].set(x)
    xs = jnp.stack([xp[:, :CD], xp[:, CD:]])

    degs_p, degd_p = _deg_call(srcp32, dstp32)
    ns80, nd80 = _norm_call(
        degs_p.reshape(NW, NP // 128, 128), degd_p.reshape(NW, NP // 128, 128)
    )
    nscol = ns80.reshape(NP, 1)
    ndcol = nd80.reshape(NP, 1)

    b1r, g1r, be1r = b1.reshape(1, D), g1.reshape(1, D), be1.reshape(1, D)
    b2r, g2r, be2r = b2.reshape(1, D), g2.reshape(1, D), be2.reshape(1, D)

    xss = _scale_table(xs, nscol)
    parts1 = _msg_call(xss.reshape(NC * NP, CD), srcp16, dstp16, ewp16)
    h1s = _dense_mid(parts1, ndcol, nscol, W1, b1r, g1r, be1r)
    parts2 = _msg_call(h1s.reshape(NC * NP, CD), srcp16, dstp16, ewp16)
    out = _dense_final(parts2, ndcol, W2, b2r, g2r, be2r, xs)
    return jnp.concatenate([out[0, :N], out[1, :N]], axis=1)
